# Initial kernel scaffold; baseline (speedup 1.0000x reference)
import jax
import jax.numpy as jnp
from jax.experimental import pallas as pl


def kernel(x, edge_index, edge_attr, batch, Wf, bf, W1_0, b1_0, W2_0, W3_0, b3_0, W1_1, b1_1, W2_1, W3_1, b3_1, W1_2, b1_2, W2_2, W3_2, b3_2):
    raise NotImplementedError("write your pallas kernel here")


# speedup vs baseline: 6.6791x; 6.6791x over previous
import jax
import jax.numpy as jnp
from jax import lax
from jax.experimental import pallas as pl
from jax.experimental.pallas import tpu as pltpu
from jax.experimental.pallas import tpu_sc as plsc

NSUB = 16
NCORE = 2
LANE = 16


def _sc_scatter_layer(table, srcH, dstH, ewH, n_nodes, hc, with_s, spad):
    nch, c_sz = dstH.shape[1], dstH.shape[2]
    rpt = (n_nodes // (NSUB * 8)) * 8
    tail = n_nodes - NSUB * rpt
    nzc = next(k for k in range(1, 64)
               if rpt % k == 0 and rpt // k <= c_sz and (rpt // k) % 8 == 0)
    rz = rpt // nzc
    hvecs = hc // LANE
    stage = (2 * n_nodes * hc) <= 1_100_000

    mesh = plsc.VectorSubcoreMesh(core_axis_name="c", subcore_axis_name="s")
    out_type = [jax.ShapeDtypeStruct((NCORE, n_nodes, hc), jnp.float32)]
    if with_s:
        out_type.append(jax.ShapeDtypeStruct((NCORE, spad), jnp.float32))

    scratch = [
        pltpu.VMEM((nch, c_sz), jnp.int32),
        pltpu.VMEM((nch, c_sz), jnp.int32),
        pltpu.VMEM((nch, c_sz), jnp.float32),
        pltpu.VMEM((c_sz, hc), jnp.float32),
        pltpu.VMEM_SHARED((n_nodes, hc), jnp.float32),
        pltpu.SemaphoreType.DMA,
    ]
    if stage:
        scratch.append(pltpu.VMEM_SHARED((n_nodes, hc), jnp.float32))
    else:
        scratch.append(pltpu.VMEM((c_sz,), jnp.int32))
    if with_s:
        scratch += [
            pltpu.VMEM((spad // NSUB,), jnp.float32),
            pltpu.VMEM_SHARED((spad,), jnp.float32),
        ]

    def body(tbl, srcr, dstr, ewr, agg_o, *rest):
        rest = list(rest)
        if with_s:
            s_o = rest.pop(0)
        src_v, dst_v, ew_v, rowbuf, acc, gsem = rest[:6]
        rest = rest[6:]
        if stage:
            tbl_s = rest.pop(0)
            gix = None
        else:
            tbl_s = None
            gix = rest.pop(0)
        if with_s:
            zbs, s_sh = rest
        c = lax.axis_index("c")
        sid = lax.axis_index("s")

        pltpu.sync_copy(srcr.at[sid], src_v)
        pltpu.sync_copy(dstr.at[sid], dst_v)
        pltpu.sync_copy(ewr.at[sid], ew_v)

        if stage:
            pltpu.sync_copy(tbl.at[pl.ds(c * n_nodes + sid * rpt, rpt)],
                            tbl_s.at[pl.ds(sid * rpt, rpt)])
            if tail > 0:
                @pl.when(sid == 0)
                def _():
                    pltpu.sync_copy(tbl.at[pl.ds(c * n_nodes + NSUB * rpt, tail)],
                                    tbl_s.at[pl.ds(NSUB * rpt, tail)])

        z16 = jnp.zeros((LANE,), jnp.float32)

        def zrow(r, _):
            for h in range(hvecs):
                rowbuf[r, pl.ds(h * LANE, LANE)] = z16
            return 0
        lax.fori_loop(0, rz, zrow, 0)

        def zacc(j, _):
            pltpu.sync_copy(rowbuf.at[pl.ds(0, rz)],
                            acc.at[pl.ds(sid * rpt + j * rz, rz)])
            return 0
        lax.fori_loop(0, nzc, zacc, 0)
        if tail > 0:
            @pl.when(sid == 0)
            def _():
                pltpu.sync_copy(rowbuf.at[pl.ds(0, tail)],
                                acc.at[pl.ds(NSUB * rpt, tail)])

        if with_s:
            spt = spad // NSUB

            def zsrow(i, _):
                zbs[pl.ds(i * LANE, LANE)] = z16
                return 0
            lax.fori_loop(0, spt // LANE, zsrow, 0)
            pltpu.sync_copy(zbs, s_sh.at[pl.ds(sid * spt, spt)])

        plsc.subcore_barrier()

        cN = c * n_nodes

        def chunk(k, _):
            if stage:
                pltpu.async_copy(tbl_s.at[src_v.at[k]], rowbuf, gsem).wait()
            else:
                def gxi(j, _):
                    gix[pl.ds(j * LANE, LANE)] = (
                        src_v[k, pl.ds(j * LANE, LANE)] + cN)
                    return 0
                lax.fori_loop(0, c_sz // LANE, gxi, 0)
                pltpu.async_copy(tbl.at[gix], rowbuf, gsem).wait()

            def scale(j, _):
                ewv = ew_v[k, pl.ds(j * LANE, LANE)]
                for i in range(LANE):
                    w = ewv[i]
                    ei = j * LANE + i
                    for h in range(hvecs):
                        sl = pl.ds(h * LANE, LANE)
                        rowbuf[ei, sl] = rowbuf[ei, sl] * w
                return 0
            lax.fori_loop(0, c_sz // LANE, scale, 0)

            pltpu.sync_copy(rowbuf, acc.at[dst_v.at[k]], add=True)
            if with_s:
                @pl.when(lax.rem(k, 2) == c)
                def _():
                    pltpu.sync_copy(ew_v.at[k], s_sh.at[dst_v.at[k]], add=True)
            return 0
        lax.fori_loop(0, nch, chunk, 0)

        plsc.subcore_barrier()

        pltpu.sync_copy(acc.at[pl.ds(sid * rpt, rpt)],
                        agg_o.at[c, pl.ds(sid * rpt, rpt)])
        if tail > 0:
            @pl.when(sid == 0)
            def _():
                pltpu.sync_copy(acc.at[pl.ds(NSUB * rpt, tail)],
                                agg_o.at[c, pl.ds(NSUB * rpt, tail)])
        if with_s:
            @pl.when(sid == 0)
            def _():
                pltpu.sync_copy(s_sh, s_o.at[c])

    fn = pl.kernel(body, out_type=tuple(out_type), mesh=mesh,
                   scratch_types=tuple(scratch),
                   compiler_params=pltpu.CompilerParams(use_tc_tiling_on_sc=False))
    return fn(table, srcH, dstH, ewH)


def _elu(x):
    return jnp.where(x > 0, x, jnp.exp(x) - 1.0)


def _edge_weights(edge_attr, wf_row, bf):
    e = edge_attr.shape[0]
    eb = 8000

    def body(ea_ref, wf_ref, bf_ref, o_ref):
        o_ref[...] = (jnp.sum(ea_ref[...] * wf_ref[...], axis=1, keepdims=True)
                      + bf_ref[...])

    return pl.pallas_call(
        body,
        grid=(e // eb,),
        in_specs=[
            pl.BlockSpec((eb, edge_attr.shape[1]), lambda i: (i, 0)),
            pl.BlockSpec((1, edge_attr.shape[1]), lambda i: (0, 0)),
            pl.BlockSpec((1, 1), lambda i: (0, 0)),
        ],
        out_specs=pl.BlockSpec((eb, 1), lambda i: (i, 0)),
        out_shape=jax.ShapeDtypeStruct((e, 1), jnp.float32),
    )(edge_attr, wf_row, bf)


def _proj0(x, w1, b1, w2, w3, b3):
    n, d = x.shape
    hn = w1.shape[1]
    hc = hn // 2
    r = 2000

    def body(x_ref, w1_ref, b1_ref, w2_ref, w3_ref, b3_ref,
             ah_ref, b_ref, c_ref):
        xb = x_ref[...]
        a = jnp.dot(xb, w1_ref[...], preferred_element_type=jnp.float32) + b1_ref[...]
        ah_ref[0] = a[:, :hc]
        ah_ref[1] = a[:, hc:]
        b_ref[...] = jnp.dot(xb, w2_ref[...], preferred_element_type=jnp.float32)
        c_ref[...] = (jnp.dot(xb, w3_ref[...], preferred_element_type=jnp.float32)
                      + b3_ref[...])

    return pl.pallas_call(
        body,
        grid=(n // r,),
        in_specs=[
            pl.BlockSpec((r, d), lambda i: (i, 0)),
            pl.BlockSpec((d, hn), lambda i: (0, 0)),
            pl.BlockSpec((1, hn), lambda i: (0, 0)),
            pl.BlockSpec((d, hn), lambda i: (0, 0)),
            pl.BlockSpec((d, hn), lambda i: (0, 0)),
            pl.BlockSpec((1, hn), lambda i: (0, 0)),
        ],
        out_specs=[
            pl.BlockSpec((2, r, hc), lambda i: (0, i, 0)),
            pl.BlockSpec((r, hn), lambda i: (i, 0)),
            pl.BlockSpec((r, hn), lambda i: (i, 0)),
        ],
        out_shape=[
            jax.ShapeDtypeStruct((2, n, hc), jnp.float32),
            jax.ShapeDtypeStruct((n, hn), jnp.float32),
            jax.ShapeDtypeStruct((n, hn), jnp.float32),
        ],
    )(x, w1, b1, w2, w3, b3)


def _combine(aggh, sA, sB, bmat, cmat, w1, b1, w2, w3, b3):
    n, hin = bmat.shape
    hc = hin // 2
    hn = w1.shape[1]
    hcn = hn // 2
    r = 2000

    def body(ag_ref, sa_ref, sb_ref, b_ref, c_ref,
             w1_ref, b1_ref, w2_ref, w3_ref, b3_ref,
             ah_ref, bn_ref, cn_ref):
        st = sa_ref[...] + sb_ref[...]
        hL = _elu(ag_ref[0] - b_ref[:, :hc] * st + c_ref[:, :hc])
        hR = _elu(ag_ref[1] - b_ref[:, hc:] * st + c_ref[:, hc:])
        w1v = w1_ref[...]
        a = (jnp.dot(hL, w1v[:hc], preferred_element_type=jnp.float32)
             + jnp.dot(hR, w1v[hc:], preferred_element_type=jnp.float32)
             + b1_ref[...])
        ah_ref[0] = a[:, :hcn]
        ah_ref[1] = a[:, hcn:]
        w2v = w2_ref[...]
        bn_ref[...] = (jnp.dot(hL, w2v[:hc], preferred_element_type=jnp.float32)
                       + jnp.dot(hR, w2v[hc:], preferred_element_type=jnp.float32))
        w3v = w3_ref[...]
        cn_ref[...] = (jnp.dot(hL, w3v[:hc], preferred_element_type=jnp.float32)
                       + jnp.dot(hR, w3v[hc:], preferred_element_type=jnp.float32)
                       + b3_ref[...])

    return pl.pallas_call(
        body,
        grid=(n // r,),
        in_specs=[
            pl.BlockSpec((2, r, hc), lambda i: (0, i, 0)),
            pl.BlockSpec((r, 1), lambda i: (i, 0)),
            pl.BlockSpec((r, 1), lambda i: (i, 0)),
            pl.BlockSpec((r, hin), lambda i: (i, 0)),
            pl.BlockSpec((r, hin), lambda i: (i, 0)),
            pl.BlockSpec((hin, hn), lambda i: (0, 0)),
            pl.BlockSpec((1, hn), lambda i: (0, 0)),
            pl.BlockSpec((hin, hn), lambda i: (0, 0)),
            pl.BlockSpec((hin, hn), lambda i: (0, 0)),
            pl.BlockSpec((1, hn), lambda i: (0, 0)),
        ],
        out_specs=[
            pl.BlockSpec((2, r, hcn), lambda i: (0, i, 0)),
            pl.BlockSpec((r, hn), lambda i: (i, 0)),
            pl.BlockSpec((r, hn), lambda i: (i, 0)),
        ],
        out_shape=[
            jax.ShapeDtypeStruct((2, n, hcn), jnp.float32),
            jax.ShapeDtypeStruct((n, hn), jnp.float32),
            jax.ShapeDtypeStruct((n, hn), jnp.float32),
        ],
    )(aggh, sA, sB, bmat, cmat, w1, b1, w2, w3, b3)


def _final_segmax(aggh, sA, sB, bmat, cmat, batch2d, n_seg):
    n, hin = bmat.shape
    hc = hin // 2
    r = 2000

    def body(ag_ref, sa_ref, sb_ref, b_ref, c_ref, bat_ref, o_ref):
        i = pl.program_id(0)
        st = sa_ref[...] + sb_ref[...]
        hL = _elu(ag_ref[0] - b_ref[:, :hc] * st + c_ref[:, :hc])
        hR = _elu(ag_ref[1] - b_ref[:, hc:] * st + c_ref[:, hc:])
        bat = bat_ref[...]

        @pl.when(i == 0)
        def _():
            o_ref[...] = jnp.full((n_seg, hin), -jnp.inf, jnp.float32)

        def g_body(g, _):
            mask = bat == g
            mL = jnp.max(jnp.where(mask, hL, -jnp.inf), axis=0, keepdims=True)
            mR = jnp.max(jnp.where(mask, hR, -jnp.inf), axis=0, keepdims=True)
            o_ref[pl.ds(g, 1), :hc] = jnp.maximum(o_ref[pl.ds(g, 1), :hc], mL)
            o_ref[pl.ds(g, 1), hc:] = jnp.maximum(o_ref[pl.ds(g, 1), hc:], mR)
            return 0
        lax.fori_loop(0, n_seg, g_body, 0)

    return pl.pallas_call(
        body,
        grid=(n // r,),
        in_specs=[
            pl.BlockSpec((2, r, hc), lambda i: (0, i, 0)),
            pl.BlockSpec((r, 1), lambda i: (i, 0)),
            pl.BlockSpec((r, 1), lambda i: (i, 0)),
            pl.BlockSpec((r, hin), lambda i: (i, 0)),
            pl.BlockSpec((r, hin), lambda i: (i, 0)),
            pl.BlockSpec((r, 1), lambda i: (i, 0)),
        ],
        out_specs=pl.BlockSpec((n_seg, hin), lambda i: (0, 0)),
        out_shape=jax.ShapeDtypeStruct((n_seg, hin), jnp.float32),
    )(aggh, sA, sB, bmat, cmat, batch2d)


def kernel(x, edge_index, edge_attr, batch, Wf, bf,
           W1_0, b1_0, W2_0, W3_0, b3_0,
           W1_1, b1_1, W2_1, W3_1, b3_1,
           W1_2, b1_2, W2_2, W3_2, b3_2):
    n, d = x.shape
    e = edge_index.shape[1]
    n_seg = 64
    c_sz = 400
    et = e // NSUB
    nch = et // c_sz
    spad = ((n + NSUB * LANE - 1) // (NSUB * LANE)) * NSUB * LANE

    src = edge_index[0]
    dst = edge_index[1]

    ew = _edge_weights(edge_attr, Wf.reshape(1, -1), bf.reshape(1, 1))
    ewH = ew.reshape(NSUB, nch, c_sz)
    srcH = src.reshape(NSUB, nch, c_sz)
    dstH = dst.reshape(NSUB, nch, c_sz)

    a0h, b0, c0 = _proj0(x, W1_0, b1_0.reshape(1, -1), W2_0, W3_0,
                         b3_0.reshape(1, -1))
    h0 = W1_0.shape[1]
    agg0, s2 = _sc_scatter_layer(a0h.reshape(2 * n, h0 // 2), srcH, dstH, ewH,
                                 n, h0 // 2, True, spad)
    sA = s2[0, :n].reshape(n, 1)
    sB = s2[1, :n].reshape(n, 1)

    a1h, b1v, c1v = _combine(agg0, sA, sB, b0, c0, W1_1, b1_1.reshape(1, -1),
                             W2_1, W3_1, b3_1.reshape(1, -1))
    h1 = W1_1.shape[1]
    (agg1,) = _sc_scatter_layer(a1h.reshape(2 * n, h1 // 2), srcH, dstH, ewH,
                                n, h1 // 2, False, spad)

    a2h, b2v, c2v = _combine(agg1, sA, sB, b1v, c1v, W1_2, b1_2.reshape(1, -1),
                             W2_2, W3_2, b3_2.reshape(1, -1))
    h2 = W1_2.shape[1]
    (agg2,) = _sc_scatter_layer(a2h.reshape(2 * n, h2 // 2), srcH, dstH, ewH,
                                n, h2 // 2, False, spad)

    return _final_segmax(agg2, sA, sB, b2v, c2v, batch.reshape(n, 1), n_seg)
